# baseline (speedup 1.0000x reference)
import functools

import jax
import jax.numpy as jnp
from jax import lax
from jax.experimental import pallas as pl
from jax.experimental.pallas import tpu as pltpu
from jax.experimental.pallas import tpu_sc as plsc

N_NODES = 50000
D_IN = 512
D_OUT = 128
NUM_GRAPHS = 512

NC, NS = 2, 16
NW = NC * NS
RS, CG = 8, 4
CW = D_IN // CG
SHARD = 6272
LSHARD = N_NODES - 7 * SHARD
CHUNK = 128
NCOMMON = 47
NFULL = 49
LTAIL = LSHARD - NCOMMON * CHUNK

CNT_CHUNK = 2000
CNT_K = N_NODES // CNT_CHUNK


def _sc_segment_sums(x, ids2d):
    mesh = plsc.VectorSubcoreMesh(core_axis_name="c", subcore_axis_name="s")

    @functools.partial(
        pl.kernel,
        out_type=jax.ShapeDtypeStruct((RS, CG, NUM_GRAPHS, CW), jnp.float32),
        mesh=mesh,
        compiler_params=pltpu.CompilerParams(needs_layout_passes=False),
        scratch_types=[
            pltpu.VMEM((CHUNK // 16, 16), jnp.int32),
            pltpu.VMEM((CHUNK // 16, 16), jnp.int32),
            pltpu.VMEM((CHUNK, CW), jnp.float32),
            pltpu.VMEM((CHUNK, CW), jnp.float32),
            pltpu.VMEM((NUM_GRAPHS, CW), jnp.float32),
            pltpu.SemaphoreType.DMA,
            pltpu.SemaphoreType.DMA,
            pltpu.SemaphoreType.DMA,
            pltpu.SemaphoreType.DMA,
        ],
    )
    def sums_kernel(x_hbm, ids_hbm, out, idx0, idx1, xb0, xb1, acc,
                    si0, si1, sx0, sx1):
        c = lax.axis_index("c")
        s = lax.axis_index("s")
        w = s * NC + c
        rs = w // CG
        cg = w % CG
        col = cg * CW
        base = rs * SHARD
        gbase = rs * (SHARD // 16)

        zeros16 = jnp.zeros((16,), jnp.float32)

        def zrow(i, _):
            for j in range(CW // 16):
                acc[i, pl.ds(j * 16, 16)] = zeros16
            return 0

        lax.fori_loop(0, NUM_GRAPHS, zrow, 0)

        bufs = [(idx0, xb0, si0, sx0), (idx1, xb1, si1, sx1)]

        def start_gather(slot, i, nrows):
            idx_v, xb, si, sx = bufs[slot]
            pltpu.async_copy(
                ids_hbm.at[pl.ds(gbase + i * (CHUNK // 16), CHUNK // 16)],
                idx_v, si)
            pltpu.async_copy(
                x_hbm.at[pl.ds(base + i * CHUNK, nrows), pl.ds(col, CW)],
                xb.at[pl.ds(0, nrows)], sx)

        def wait_gather(slot, i, nrows):
            idx_v, xb, si, sx = bufs[slot]
            pltpu.make_async_copy(
                ids_hbm.at[pl.ds(gbase + i * (CHUNK // 16), CHUNK // 16)],
                idx_v, si).wait()
            pltpu.make_async_copy(
                x_hbm.at[pl.ds(base + i * CHUNK, nrows), pl.ds(col, CW)],
                xb.at[pl.ds(0, nrows)], sx).wait()

        iotas = [
            jax.lax.broadcasted_iota(jnp.int32, (16,), 0) + 16 * j
            for j in range(CW // 16)
        ]
        lane0 = jax.lax.broadcasted_iota(jnp.int32, (16,), 0) * 0
        dnums = lax.GatherDimensionNumbers(
            offset_dims=(), collapsed_slice_dims=(0,), start_index_map=(0,))

        def accumulate(slot, ngroups):
            idx_v, xb, _, _ = bufs[slot]

            def grp(g, _):
                ids_vec = idx_v[g, :]
                for r in range(16):
                    idb = lax.gather(
                        ids_vec, (lane0 + r).reshape(16, 1), dnums,
                        slice_sizes=(1,),
                        mode=lax.GatherScatterMode.PROMISE_IN_BOUNDS)
                    row = g * 16 + r
                    for j in range(CW // 16):
                        xv = xb[row, pl.ds(j * 16, 16)]
                        plsc.addupdate_scatter(acc, [idb, iotas[j]], xv)
                return 0

            lax.fori_loop(0, ngroups, grp, 0)

        start_gather(0, 0, CHUNK)
        start_gather(1, 1, CHUNK)

        def pair(t, _):
            i0 = 2 * t
            i1 = 2 * t + 1

            @pl.when(i0 < NCOMMON)
            def _():
                wait_gather(0, i0, CHUNK)
                accumulate(0, CHUNK // 16)

                @pl.when(i0 + 2 < NCOMMON)
                def _():
                    start_gather(0, i0 + 2, CHUNK)

            @pl.when(i1 < NCOMMON)
            def _():
                wait_gather(1, i1, CHUNK)
                accumulate(1, CHUNK // 16)

                @pl.when(i1 + 2 < NCOMMON)
                def _():
                    start_gather(1, i1 + 2, CHUNK)

            return 0

        lax.fori_loop(0, (NCOMMON + 1) // 2, pair, 0)

        @pl.when(rs != RS - 1)
        def _():
            for i in (NCOMMON, NCOMMON + 1):
                start_gather(0, i, CHUNK)
                wait_gather(0, i, CHUNK)
                accumulate(0, CHUNK // 16)

        @pl.when(rs == RS - 1)
        def _():
            start_gather(0, NCOMMON, LTAIL)
            wait_gather(0, NCOMMON, LTAIL)
            accumulate(0, LTAIL // 16)

        pltpu.sync_copy(acc, out.at[rs, cg])

    return sums_kernel(x, ids2d)


def _count_body(ids_ref, cnt_ref):
    k = pl.program_id(0)

    @pl.when(k == 0)
    def _():
        cnt_ref[...] = jnp.zeros_like(cnt_ref)

    ids = ids_ref[0, 0, :].reshape(CNT_CHUNK, 1)
    gids = jax.lax.broadcasted_iota(jnp.int32, (1, NUM_GRAPHS), 1)
    one_hot = (ids == gids).astype(jnp.float32)
    cnt_ref[0:1, :] += jnp.sum(one_hot, axis=0, keepdims=True)


def _finish_body(p_ref, c_ref, w_ref, b_ref, o_ref):
    counts = c_ref[0:1, :].reshape(NUM_GRAPHS, 1)
    inv = 1.0 / jnp.maximum(counts, 1.0)
    blocks = []
    for cg in range(CG):
        ssum = p_ref[0, cg]
        for rs in range(1, RS):
            ssum = ssum + p_ref[rs, cg]
        blocks.append(ssum)
    emb = jnp.concatenate(blocks, axis=1) * inv
    o_ref[...] = (
        jnp.dot(emb, w_ref[...], preferred_element_type=jnp.float32)
        + b_ref[...]
    )


def kernel(x, batch_ids, y, W, b):
    ids = batch_ids.astype(jnp.int32)
    ids3d = ids.reshape(CNT_K, 1, CNT_CHUNK)
    counts = pl.pallas_call(
        _count_body,
        grid=(CNT_K,),
        in_specs=[pl.BlockSpec((1, 1, CNT_CHUNK), lambda k: (k, 0, 0))],
        out_specs=pl.BlockSpec((8, NUM_GRAPHS), lambda k: (0, 0)),
        out_shape=jax.ShapeDtypeStruct((8, NUM_GRAPHS), jnp.float32),
    )(ids3d)
    ids_pad = jnp.pad(ids, (0, 3128 * 16 - N_NODES))
    partial = _sc_segment_sums(x, ids_pad.reshape(3128, 16))
    pred = pl.pallas_call(
        _finish_body,
        out_shape=jax.ShapeDtypeStruct((NUM_GRAPHS, D_OUT), jnp.float32),
    )(partial, counts, W, b.reshape(1, D_OUT))
    return (pred, y)
